# baseline (speedup 1.0000x reference)
import functools

import numpy as np
import jax
import jax.numpy as jnp
from jax import lax
from jax.experimental import pallas as pl
from jax.experimental.pallas import tpu as pltpu
from jax.experimental.pallas import tpu_sc as plsc

H = 224
HH = H // 2
OW = 4 * H
B = 4
C = 192
NPLANES = B * C
OUT_PLANE = HH * OW
NWORKERS = 32
PER_WORKER = NPLANES // NWORKERS
CROWS = 8
NCHUNK = HH // CROWS
CHUNK = CROWS * OW
ROWVREG = OW // 32
IDXWORDS = OUT_PLANE // 2
IDXCHUNK = IDXWORDS // NCHUNK
LSTRIDE = H + 1
LINWORDS = H * LSTRIDE


def _build_src_map() -> np.ndarray:
    lmid = (H - 1) // 2
    r = np.arange(HH)[:, None]
    j = np.arange(OW)[None, :]
    i = lmid - r
    el = 2 * r + 1
    hi = i + el
    b1 = 3 * i + el
    b2 = 3 * i + 2 * el
    b3 = 7 * i + 3 * el
    b4 = 7 * i + 4 * el
    body = 4 * H - 4
    k = 5 * i + 2 * el + hi
    src_a = i * H + np.clip(j - body * (j >= b4), i, hi)
    src_b = hi * H + np.clip(k - j, i, hi)
    src_cr = (j - (2 * i + el)) * H + hi
    src_cl = (body - j) * H + i
    src = np.where(j < b1, src_a,
          np.where(j < b2, src_cr,
          np.where(j < b3, src_b,
          np.where(j < b4, src_cl, src_a))))
    return src.reshape(-1)


def _build_packed_idx() -> np.ndarray:
    src = _build_src_map()
    phys = ((src // H) * LSTRIDE + src % H).astype(np.uint32).reshape(-1, 2, 16)
    packed = phys[:, 0, :] | (phys[:, 1, :] << 16)
    return packed.reshape(-1).view(np.int32)


_IDX_PACKED = _build_packed_idx()


def _sqrl_gather_body(x_hbm, idx_hbm, out_hbm, plane2_v, plane1_v, idxb_v,
                      outb_v, insem, isem, osem):
    wid = lax.axis_index("s") * 2 + lax.axis_index("c")

    def plane_dma(p, sync=False):
        pb = lax.div(p, C)
        pc = lax.rem(p, C)
        return pltpu.async_copy(x_hbm.at[pb, pc], plane2_v, insem)

    def idx_prefetch(c):
        pltpu.async_copy(idx_hbm.at[pl.ds(c * IDXCHUNK, IDXCHUNK)],
                         idxb_v.at[lax.rem(c, 2)], isem.at[lax.rem(c, 2)])

    def wait_input():
        pltpu.make_async_copy(x_hbm.at[0, 0], plane2_v, insem).wait()

    def wait_idx(buf):
        pltpu.make_async_copy(idx_hbm.at[pl.ds(0, IDXCHUNK)],
                              idxb_v.at[buf], isem.at[buf]).wait()

    def drain_out(buf):
        pltpu.make_async_copy(out_hbm.at[0, 0, pl.ds(0, CROWS), :],
                              outb_v.at[buf], osem.at[buf]).wait()

    plane_dma(wid * PER_WORKER)
    idx_prefetch(0)

    def plane_body(pi, carry):
        p = wid * PER_WORKER + pi
        pb = lax.div(p, C)
        pc = lax.rem(p, C)
        wait_input()

        @plsc.parallel_loop(0, H, unroll=2)
        def copy_row(row):
            for k in range(H // 16):
                plane1_v[pl.ds(row * LSTRIDE + k * 16, 16)] = (
                    plane2_v[row, pl.ds(k * 16, 16)])

        plane_dma(jnp.minimum(p + 1, NPLANES - 1))

        def chunk_body(c, carry2):
            buf = lax.rem(c, 2)
            wait_idx(buf)
            idx_prefetch(lax.rem(c + 1, NCHUNK))

            @pl.when(c >= 2)
            def _():
                drain_out(buf)

            @plsc.parallel_loop(0, CROWS, unroll=4)
            def vbody(row):
                base = row * (ROWVREG * 16)
                for kk in range(ROWVREG):
                    vp = idxb_v[buf, pl.ds(base + kk * 16, 16)]
                    lo = jnp.bitwise_and(vp, 0xFFFF)
                    hi = lax.shift_right_logical(vp, 16)
                    outb_v[buf, row, pl.ds(kk * 32, 16)] = (
                        plsc.load_gather(plane1_v, [lo]))
                    outb_v[buf, row, pl.ds(kk * 32 + 16, 16)] = (
                        plsc.load_gather(plane1_v, [hi]))

            pltpu.async_copy(
                outb_v.at[buf],
                out_hbm.at[pb, pc, pl.ds(c * CROWS, CROWS), :],
                osem.at[buf])
            return carry2

        lax.fori_loop(0, NCHUNK, chunk_body, 0)
        drain_out(0)
        drain_out(1)
        return carry

    lax.fori_loop(0, PER_WORKER, plane_body, 0)
    wait_input()
    wait_idx(0)


@functools.cache
def _sqrl_gather():
    mesh = plsc.VectorSubcoreMesh(core_axis_name="c", subcore_axis_name="s")
    return pl.kernel(
        _sqrl_gather_body,
        out_type=jax.ShapeDtypeStruct((B, C, HH, OW), jnp.float32),
        mesh=mesh,
        scratch_types=[
            pltpu.VMEM((H, H), jnp.float32),
            pltpu.VMEM((LINWORDS,), jnp.float32),
            pltpu.VMEM((2, IDXCHUNK), jnp.int32),
            pltpu.VMEM((2, CROWS, OW), jnp.float32),
            pltpu.SemaphoreType.DMA,
            pltpu.SemaphoreType.DMA((2,)),
            pltpu.SemaphoreType.DMA((2,)),
        ],
        compiler_params=pltpu.CompilerParams(
            needs_layout_passes=False, disable_bounds_checks=True),
    )


def kernel(x):
    return _sqrl_gather()(x, jnp.asarray(_IDX_PACKED))


# speedup vs baseline: 1.0845x; 1.0845x over previous
import functools

import numpy as np
import jax
import jax.numpy as jnp
from jax import lax
from jax.experimental import pallas as pl
from jax.experimental.pallas import tpu as pltpu
from jax.experimental.pallas import tpu_sc as plsc

H = 224
HH = H // 2
OW = 4 * H
B = 4
C = 192
NPLANES = B * C
OUT_PLANE = HH * OW
NWORKERS = 32
PER_WORKER = NPLANES // NWORKERS
CROWS = 8
NCHUNK = HH // CROWS
CHUNK = CROWS * OW
ROWVREG = OW // 32
IDXWORDS = OUT_PLANE // 2
IDXCHUNK = IDXWORDS // NCHUNK
LSTRIDE = H + 1
LINWORDS = H * LSTRIDE


def _build_src_map() -> np.ndarray:
    lmid = (H - 1) // 2
    r = np.arange(HH)[:, None]
    j = np.arange(OW)[None, :]
    i = lmid - r
    el = 2 * r + 1
    hi = i + el
    b1 = 3 * i + el
    b2 = 3 * i + 2 * el
    b3 = 7 * i + 3 * el
    b4 = 7 * i + 4 * el
    body = 4 * H - 4
    k = 5 * i + 2 * el + hi
    src_a = i * H + np.clip(j - body * (j >= b4), i, hi)
    src_b = hi * H + np.clip(k - j, i, hi)
    src_cr = (j - (2 * i + el)) * H + hi
    src_cl = (body - j) * H + i
    src = np.where(j < b1, src_a,
          np.where(j < b2, src_cr,
          np.where(j < b3, src_b,
          np.where(j < b4, src_cl, src_a))))
    return src.reshape(-1)


def _build_packed_idx() -> np.ndarray:
    src = _build_src_map()
    phys = ((src // H) * LSTRIDE + src % H).astype(np.uint32).reshape(-1, 2, 16)
    packed = phys[:, 0, :] | (phys[:, 1, :] << 16)
    return packed.reshape(-1).view(np.int32)


_IDX_PACKED = _build_packed_idx()


def _sqrl_gather_body(x_hbm, idx_hbm, out_hbm, plane2_v, plane1_v, idxb_v,
                      outb_v, insem, isem, osem):
    wid = lax.axis_index("s") * 2 + lax.axis_index("c")

    def plane_dma(p):
        pb = lax.div(p, C)
        pc = lax.rem(p, C)
        return pltpu.async_copy(x_hbm.at[pb, pc], plane2_v, insem)

    def idx_prefetch(c):
        pltpu.async_copy(idx_hbm.at[pl.ds(c * IDXCHUNK, IDXCHUNK)],
                         idxb_v.at[lax.rem(c, 2)], isem.at[lax.rem(c, 2)])

    def wait_input():
        pltpu.make_async_copy(x_hbm.at[0, 0], plane2_v, insem).wait()

    def wait_idx(buf):
        pltpu.make_async_copy(idx_hbm.at[pl.ds(0, IDXCHUNK)],
                              idxb_v.at[buf], isem.at[buf]).wait()

    def drain_out(buf):
        pltpu.make_async_copy(out_hbm.at[0, 0, pl.ds(0, CROWS), :],
                              outb_v.at[buf], osem.at[buf]).wait()

    plane_dma(wid * PER_WORKER)
    idx_prefetch(0)

    def plane_body(pi, carry):
        p = wid * PER_WORKER + pi
        pb = lax.div(p, C)
        pc = lax.rem(p, C)
        wait_input()

        @plsc.parallel_loop(0, H, unroll=2)
        def copy_row(row):
            for k in range(H // 16):
                plane1_v[pl.ds(row * LSTRIDE + k * 16, 16)] = (
                    plane2_v[row, pl.ds(k * 16, 16)])

        plane_dma(jnp.minimum(p + 1, NPLANES - 1))

        def chunk_body(c, carry2):
            buf = lax.rem(c, 2)
            wait_idx(buf)
            idx_prefetch(lax.rem(c + 1, NCHUNK))

            @pl.when(c >= 2)
            def _():
                drain_out(buf)

            @plsc.parallel_loop(0, CROWS, unroll=2)
            def vbody(row):
                base = row * (ROWVREG * 16)
                for kk in range(ROWVREG):
                    vp = idxb_v[buf, pl.ds(base + kk * 16, 16)]
                    lo = jnp.bitwise_and(vp, 0xFFFF)
                    hi = lax.shift_right_logical(vp, 16)
                    outb_v[buf, row, pl.ds(kk * 32, 16)] = (
                        plsc.load_gather(plane1_v, [lo]))
                    outb_v[buf, row, pl.ds(kk * 32 + 16, 16)] = (
                        plsc.load_gather(plane1_v, [hi]))

            pltpu.async_copy(
                outb_v.at[buf],
                out_hbm.at[pb, pc, pl.ds(c * CROWS, CROWS), :],
                osem.at[buf])
            return carry2

        lax.fori_loop(0, NCHUNK, chunk_body, 0)
        drain_out(0)
        drain_out(1)
        return carry

    lax.fori_loop(0, PER_WORKER, plane_body, 0)
    wait_input()
    wait_idx(0)


@functools.cache
def _sqrl_gather():
    mesh = plsc.VectorSubcoreMesh(core_axis_name="c", subcore_axis_name="s")
    return pl.kernel(
        _sqrl_gather_body,
        out_type=jax.ShapeDtypeStruct((B, C, HH, OW), jnp.float32),
        mesh=mesh,
        scratch_types=[
            pltpu.VMEM((H, H), jnp.float32),
            pltpu.VMEM((LINWORDS,), jnp.float32),
            pltpu.VMEM((2, IDXCHUNK), jnp.int32),
            pltpu.VMEM((2, CROWS, OW), jnp.float32),
            pltpu.SemaphoreType.DMA,
            pltpu.SemaphoreType.DMA((2,)),
            pltpu.SemaphoreType.DMA((2,)),
        ],
        compiler_params=pltpu.CompilerParams(
            needs_layout_passes=False, disable_bounds_checks=True),
    )


def kernel(x):
    return _sqrl_gather()(x, jnp.asarray(_IDX_PACKED))
